# Initial kernel scaffold; baseline (speedup 1.0000x reference)
import jax
import jax.numpy as jnp
from jax.experimental import pallas as pl


def kernel(input1, input2):
    raise NotImplementedError("write your pallas kernel here")


# speedup vs baseline: 7.8547x; 7.8547x over previous
import functools

import jax
import jax.numpy as jnp
from jax import lax
from jax.experimental import pallas as pl
from jax.experimental.pallas import tpu as pltpu
from jax.experimental.pallas import tpu_sc as plsc

B = 16
N = 4096
K = 20
D = 128

NC = 2
NS = 16
NW = NC * NS

QW = (B * N) // NW
C = 16
NCH = QW // C
G = 64
NG = (C * K) // G


def _body(table, idxf, out, idx_v, rows_v, out_v, sem):
    wid = lax.axis_index("s") * NC + lax.axis_index("c")
    b = wid // (NW // B)
    boff = b * N
    base_q = wid * QW

    def chunk(t, carry):
        q0 = base_q + t * C
        pltpu.sync_copy(idxf.at[pl.ds(q0 * K, C * K)], idx_v)
        for j in range((C * K) // 16):
            sl = pl.ds(j * 16, 16)
            idx_v[sl] = idx_v[sl] + boff
        copies = [
            pltpu.async_copy(table.at[idx_v.at[pl.ds(g * G, G)]],
                             rows_v.at[pl.ds(g * G, G)], sem)
            for g in range(NG)
        ]
        for cp in copies:
            cp.wait()

        def one_q(q, carry2):
            r = q * K
            for db in range(D // 16):
                sl = pl.ds(db * 16, 16)
                acc = rows_v[r, sl]
                for k in range(1, K):
                    acc = jnp.maximum(acc, rows_v[r + k, sl])
                out_v[q, sl] = acc
            return carry2

        lax.fori_loop(0, C, one_q, 0, unroll=False)
        pltpu.sync_copy(out_v, out.at[pl.ds(q0, C)])
        return carry

    lax.fori_loop(0, NCH, chunk, 0, unroll=False)


@jax.jit
def _launch(table, idxf):
    mesh = plsc.VectorSubcoreMesh(core_axis_name="c", subcore_axis_name="s")
    return pl.kernel(
        _body,
        out_type=jax.ShapeDtypeStruct((B * N, D), jnp.float32),
        mesh=mesh,
        scratch_types=[
            pltpu.VMEM((C * K,), jnp.int32),
            pltpu.VMEM((C * K, D), jnp.float32),
            pltpu.VMEM((C, D), jnp.float32),
            pltpu.SemaphoreType.DMA,
        ],
    )(table, idxf)


def kernel(input1, input2):
    table = input1.reshape(B * N, D)
    idxf = input2.reshape(B * N * K)
    out = _launch(table, idxf)
    return out.reshape(B, N, D)


# speedup vs baseline: 15.4854x; 1.9715x over previous
import jax
import jax.numpy as jnp
from jax import lax
from jax.experimental import pallas as pl
from jax.experimental.pallas import tpu as pltpu
from jax.experimental.pallas import tpu_sc as plsc

B = 16
N = 4096
K = 20
D = 128
W = 128

NC = 2
NS = 16
NW = NC * NS

QW = (B * N) // NW
C = 16
NCH = QW // C
G = 80
NG = (C * K) // G


def _body(table, idxf, out, idx_v, rows_v, out_v, isem, gsem, osem):
    wid = lax.axis_index("s") * NC + lax.axis_index("c")
    b = wid // (NW // B)
    boff = b * N
    base_q = wid * QW

    def idx_copy(t, s):
        q0 = base_q + t * C
        return pltpu.make_async_copy(idxf.at[pl.ds(q0 * K, C * K)],
                                     idx_v.at[pl.ds(s * C * K, C * K)],
                                     isem.at[s])

    def fire_gathers(s):
        o = s * C * K
        for j in range((C * K) // 16):
            sl = pl.ds(o + j * 16, 16)
            idx_v[sl] = idx_v[sl] + boff
        for g in range(NG):
            pltpu.async_copy(table.at[idx_v.at[pl.ds(o + g * G, G)]],
                             rows_v.at[s, pl.ds(g * G, G)], gsem.at[s])

    def wait_gathers(s):
        o = s * C * K
        for g in range(NG):
            pltpu.make_async_copy(table.at[idx_v.at[pl.ds(o + g * G, G)]],
                                  rows_v.at[s, pl.ds(g * G, G)],
                                  gsem.at[s]).wait()

    def out_copy(t, s):
        q0 = base_q + t * C
        return pltpu.make_async_copy(out_v.at[s], out.at[pl.ds(q0, C)],
                                     osem.at[s])

    idx_copy(0, 0).start()
    idx_copy(1, 1).start()
    idx_copy(0, 0).wait()
    fire_gathers(0)

    def step(t, s):
        so = 1 - s
        tn = jnp.minimum(t + 1, NCH - 1)
        tn2 = jnp.minimum(t + 2, NCH - 1)

        idx_copy(tn, so).wait()
        fire_gathers(so)

        wait_gathers(s)
        idx_copy(tn2, s).start()

        @pl.when(t >= 2)
        def _():
            out_copy(t - 2, s).wait()

        def one_q(q, carry2):
            r = q * K
            for db in range(W // 16):
                sl = pl.ds(db * 16, 16)
                acc = rows_v[s, r, sl]
                for k in range(1, K):
                    acc = jnp.maximum(acc, rows_v[s, r + k, sl])
                out_v[s, q, sl] = acc
            return carry2

        lax.fori_loop(0, C, one_q, 0, unroll=False)
        out_copy(t, s).start()

    def super_step(j, carry):
        step(2 * j, 0)
        step(2 * j + 1, 1)
        return carry

    lax.fori_loop(0, NCH // 2, super_step, 0, unroll=False)

    out_copy(NCH - 2, 0).wait()
    out_copy(NCH - 1, 1).wait()


@jax.jit
def _launch(table, idxf):
    mesh = plsc.VectorSubcoreMesh(core_axis_name="c", subcore_axis_name="s")
    return pl.kernel(
        _body,
        out_type=jax.ShapeDtypeStruct((B * N, W), jnp.float32),
        mesh=mesh,
        scratch_types=[
            pltpu.VMEM((2 * C * K,), jnp.int32),
            pltpu.VMEM((2, C * K, W), jnp.float32),
            pltpu.VMEM((2, C, W), jnp.float32),
            pltpu.SemaphoreType.DMA((2,)),
            pltpu.SemaphoreType.DMA((2,)),
            pltpu.SemaphoreType.DMA((2,)),
        ],
    )(table, idxf)


def kernel(input1, input2):
    table = input1.reshape(B * N, D)
    idxf = input2.reshape(B * N * K)
    out = _launch(table, idxf)
    return out.reshape(B, N, D)
